# baseline (speedup 1.0000x reference)
import jax
import jax.numpy as jnp
from jax.experimental import pallas as pl
from jax.experimental.pallas import tpu as pltpu

_TOP_K = 8
_C = 2
_EPB = 2


def _moe_body(*refs):
    x_ref, gate_ref = refs[0], refs[1]
    w1_refs = refs[2:2 + _C]
    w3_refs = refs[2 + _C:2 + 2 * _C]
    w2_refs = refs[2 + 2 * _C:2 + 3 * _C]
    sgu_ref, sd_ref, out_ref, comb_ref = refs[2 + 3 * _C:]

    e = pl.program_id(0)
    n_e = pl.num_programs(0)
    x = x_ref[...]

    @pl.when(e == 0)
    def _router():
        logits = jnp.dot(x, gate_ref[...].T,
                         preferred_element_type=jnp.float32)
        m = jnp.max(logits, axis=-1, keepdims=True)
        p = jnp.exp(logits - m)
        p = p / jnp.sum(p, axis=-1, keepdims=True)
        lane = jax.lax.broadcasted_iota(jnp.int32, p.shape, 1)
        pm = p
        combw = jnp.zeros_like(p)
        for _ in range(_TOP_K):
            rm = jnp.max(pm, axis=-1, keepdims=True)
            eq = (pm == rm)
            first_idx = jnp.min(jnp.where(eq, lane, p.shape[1]), axis=-1,
                                keepdims=True)
            first = lane == first_idx
            combw = jnp.where(first, p, combw)
            pm = jnp.where(first, -jnp.inf, pm)
        denom = jnp.sum(combw, axis=-1, keepdims=True) + 1e-20
        comb_ref[...] = combw / denom
        gu = jnp.dot(x, sgu_ref[...].T, preferred_element_type=jnp.float32)
        si = sgu_ref.shape[0] // 2
        g = gu[:, :si]
        u = gu[:, si:]
        act = g * jax.nn.sigmoid(g) * u
        out_ref[...] = jnp.dot(act, sd_ref[...].T,
                               preferred_element_type=jnp.float32)

    lane = jax.lax.broadcasted_iota(jnp.int32, comb_ref.shape, 1)
    acc = None
    for j in range(_EPB):
        hs = []
        for c in range(_C):
            h1 = jnp.dot(x, w1_refs[c][j].T,
                         preferred_element_type=jnp.float32)
            h3 = jnp.dot(x, w3_refs[c][j].T,
                         preferred_element_type=jnp.float32)
            hs.append(h1 * jax.nn.sigmoid(h1) * h3)
        h = jnp.concatenate(hs, axis=1) if _C > 1 else hs[0]
        oe = jnp.concatenate(
            [jnp.dot(h, w2_refs[c][j].T, preferred_element_type=jnp.float32)
             for c in range(_C)], axis=1)
        wcol = jnp.sum(jnp.where(lane == e * _EPB + j, comb_ref[...], 0.0),
                       axis=1, keepdims=True)
        part = wcol * oe
        acc = part if acc is None else acc + part
    out_ref[...] += acc


def kernel(hidden_states, gate_w, w1, w3, w2, shared_gate_up, shared_down):
    b, s, d = hidden_states.shape
    x = hidden_states.reshape(-1, d)
    t = x.shape[0]
    e, ffn, _ = w1.shape
    fc = ffn // _C
    w13_specs = [pl.BlockSpec((_EPB, fc, d), lambda i, c=c: (i, c, 0))
                 for c in range(_C)]
    dc = d // _C
    w2_specs = [pl.BlockSpec((_EPB, dc, ffn), lambda i, c=c: (i, c, 0))
                for c in range(_C)]
    out = pl.pallas_call(
        _moe_body,
        grid=(e // _EPB,),
        in_specs=[
            pl.BlockSpec((t, d), lambda i: (0, 0)),
            pl.BlockSpec(gate_w.shape, lambda i: (0, 0)),
            *w13_specs,
            *w13_specs,
            *w2_specs,
            pl.BlockSpec(shared_gate_up.shape, lambda i: (0, 0)),
            pl.BlockSpec(shared_down.shape, lambda i: (0, 0)),
        ],
        out_specs=pl.BlockSpec((t, d), lambda i: (0, 0)),
        out_shape=jax.ShapeDtypeStruct((t, d), jnp.float32),
        scratch_shapes=[pltpu.VMEM((t, e), jnp.float32)],
        compiler_params=pltpu.CompilerParams(
            dimension_semantics=("arbitrary",),
            vmem_limit_bytes=100 * 1024 * 1024,
        ),
    )(x, gate_w, *([w1] * _C), *([w3] * _C), *([w2] * _C),
      shared_gate_up, shared_down)
    return out.reshape(b, s, d)


# speedup vs baseline: 1.0096x; 1.0096x over previous
import jax
import jax.numpy as jnp
from jax.experimental import pallas as pl
from jax.experimental.pallas import tpu as pltpu

_TOP_K = 8
_C = 2
_EPB = 4


def _moe_body(*refs):
    x_ref, gate_ref = refs[0], refs[1]
    w1_refs = refs[2:2 + _C]
    w3_refs = refs[2 + _C:2 + 2 * _C]
    w2_refs = refs[2 + 2 * _C:2 + 3 * _C]
    sgu_ref, sd_ref, out_ref, comb_ref = refs[2 + 3 * _C:]

    e = pl.program_id(0)
    n_e = pl.num_programs(0)
    x = x_ref[...]

    @pl.when(e == 0)
    def _router():
        logits = jnp.dot(x, gate_ref[...].T,
                         preferred_element_type=jnp.float32)
        m = jnp.max(logits, axis=-1, keepdims=True)
        p = jnp.exp(logits - m)
        p = p / jnp.sum(p, axis=-1, keepdims=True)
        lane = jax.lax.broadcasted_iota(jnp.int32, p.shape, 1)
        pm = p
        combw = jnp.zeros_like(p)
        for _ in range(_TOP_K):
            rm = jnp.max(pm, axis=-1, keepdims=True)
            eq = (pm == rm)
            first_idx = jnp.min(jnp.where(eq, lane, p.shape[1]), axis=-1,
                                keepdims=True)
            first = lane == first_idx
            combw = jnp.where(first, p, combw)
            pm = jnp.where(first, -jnp.inf, pm)
        denom = jnp.sum(combw, axis=-1, keepdims=True) + 1e-20
        comb_ref[...] = combw / denom
        out_ref[...] = jnp.zeros_like(out_ref)

    @pl.when(e == 1)
    def _shared():
        gu = jnp.dot(x, sgu_ref[...].T, preferred_element_type=jnp.float32)
        si = sgu_ref.shape[0] // 2
        g = gu[:, :si]
        u = gu[:, si:]
        act = g * jax.nn.sigmoid(g) * u
        out_ref[...] += jnp.dot(act, sd_ref[...].T,
                                preferred_element_type=jnp.float32)

    lane = jax.lax.broadcasted_iota(jnp.int32, comb_ref.shape, 1)
    acc = None
    for j in range(_EPB):
        hs = []
        for c in range(_C):
            h1 = jnp.dot(x, w1_refs[c][j].T,
                         preferred_element_type=jnp.float32)
            h3 = jnp.dot(x, w3_refs[c][j].T,
                         preferred_element_type=jnp.float32)
            hs.append(h1 * jax.nn.sigmoid(h1) * h3)
        h = jnp.concatenate(hs, axis=1) if _C > 1 else hs[0]
        oe = jnp.concatenate(
            [jnp.dot(h, w2_refs[c][j].T, preferred_element_type=jnp.float32)
             for c in range(_C)], axis=1)
        wcol = jnp.sum(jnp.where(lane == e * _EPB + j, comb_ref[...], 0.0),
                       axis=1, keepdims=True)
        part = wcol * oe
        acc = part if acc is None else acc + part
    out_ref[...] += acc


def kernel(hidden_states, gate_w, w1, w3, w2, shared_gate_up, shared_down):
    b, s, d = hidden_states.shape
    x = hidden_states.reshape(-1, d)
    t = x.shape[0]
    e, ffn, _ = w1.shape
    fc = ffn // _C
    w13_specs = [pl.BlockSpec((_EPB, fc, d), lambda i, c=c: (i, c, 0))
                 for c in range(_C)]
    dc = d // _C
    w2_specs = [pl.BlockSpec((_EPB, dc, ffn), lambda i, c=c: (i, c, 0))
                for c in range(_C)]
    out = pl.pallas_call(
        _moe_body,
        grid=(e // _EPB,),
        in_specs=[
            pl.BlockSpec((t, d), lambda i: (0, 0)),
            pl.BlockSpec(gate_w.shape, lambda i: (0, 0)),
            *w13_specs,
            *w13_specs,
            *w2_specs,
            pl.BlockSpec(shared_gate_up.shape, lambda i: (0, 0)),
            pl.BlockSpec(shared_down.shape, lambda i: (0, 0)),
        ],
        out_specs=pl.BlockSpec((t, d), lambda i: (0, 0)),
        out_shape=jax.ShapeDtypeStruct((t, d), jnp.float32),
        scratch_shapes=[pltpu.VMEM((t, e), jnp.float32)],
        compiler_params=pltpu.CompilerParams(
            dimension_semantics=("arbitrary",),
            vmem_limit_bytes=100 * 1024 * 1024,
        ),
    )(x, gate_w, *([w1] * _C), *([w3] * _C), *([w2] * _C),
      shared_gate_up, shared_down)
    return out.reshape(b, s, d)
